# baseline (speedup 1.0000x reference)
import jax
import jax.numpy as jnp
from jax.experimental import pallas as pl
from jax.experimental.pallas import tpu as pltpu

T = 2048
D = 2048
E = 16
CAP = 128
BT = 256
NB = T // BT


def _gate_kernel(x_ref, w_ref, laux_ref, cw_ref, disp_ref, cnt_ref,
                 me_acc, carry):
    i = pl.program_id(0)

    @pl.when(i == 0)
    def _init():
        me_acc[...] = jnp.zeros_like(me_acc)
        carry[...] = jnp.zeros_like(carry)

    x = x_ref[...]
    w = w_ref[...]
    logits = jax.lax.dot_general(
        x, w, (((1,), (1,)), ((), ())),
        preferred_element_type=jnp.float32)

    m = jnp.max(logits, axis=1, keepdims=True)
    p = jnp.exp(logits - m)
    s = jnp.sum(p, axis=1, keepdims=True)
    gates = p / s
    gmax = jnp.max(gates, axis=1, keepdims=True)

    colid = jax.lax.broadcasted_iota(jnp.int32, (BT, E), 1)
    idx = jnp.min(jnp.where(gates == gmax, colid, E),
                  axis=1, keepdims=True)
    mask1 = (colid == idx).astype(jnp.float32)

    me_acc[...] += jnp.sum(gates, axis=0, keepdims=True)

    r = jax.lax.broadcasted_iota(jnp.int32, (BT, BT), 0)
    c = jax.lax.broadcasted_iota(jnp.int32, (BT, BT), 1)
    tri = (r >= c).astype(jnp.float32)
    incl = carry[...] + jax.lax.dot_general(
        tri, mask1, (((1,), (0,)), ((), ())),
        preferred_element_type=jnp.float32)
    carry[...] = incl[BT - 1:BT, :]
    loc = jnp.sum((incl - 1.0) * mask1, axis=1, keepdims=True)

    kept = loc < float(CAP)
    code = jnp.where(kept, idx * CAP + loc.astype(jnp.int32), -1)
    j2 = jax.lax.broadcasted_iota(jnp.int32, (BT, E * CAP), 1)
    hit = j2 == code
    cw_ref[...] = jnp.where(hit, gmax, 0.0)
    disp_ref[...] = hit

    @pl.when(i == NB - 1)
    def _fin():
        cnt = carry[...]
        cnt_ref[...] = cnt.astype(jnp.int32)
        laux_ref[...] = jnp.sum(me_acc[...] * cnt, axis=1,
                                keepdims=True) * (E / (T * T))


def kernel(input, wg_weight):
    laux, cw, disp, cnt = pl.pallas_call(
        _gate_kernel,
        grid=(NB,),
        in_specs=[
            pl.BlockSpec((BT, D), lambda i: (i, 0)),
            pl.BlockSpec((E, D), lambda i: (0, 0)),
        ],
        out_specs=[
            pl.BlockSpec((1, 1), lambda i: (0, 0)),
            pl.BlockSpec((BT, E * CAP), lambda i: (i, 0)),
            pl.BlockSpec((BT, E * CAP), lambda i: (i, 0)),
            pl.BlockSpec((1, E), lambda i: (0, 0)),
        ],
        out_shape=[
            jax.ShapeDtypeStruct((1, 1), jnp.float32),
            jax.ShapeDtypeStruct((T, E * CAP), jnp.float32),
            jax.ShapeDtypeStruct((T, E * CAP), jnp.bool_),
            jax.ShapeDtypeStruct((1, E), jnp.int32),
        ],
        scratch_shapes=[
            pltpu.VMEM((1, E), jnp.float32),
            pltpu.VMEM((1, E), jnp.float32),
        ],
        compiler_params=pltpu.CompilerParams(
            dimension_semantics=("arbitrary",)),
    )(input.astype(jnp.float32), wg_weight.astype(jnp.float32))

    l_aux = laux.reshape(())
    combine_weights = cw.reshape(T, E, CAP)
    dispatch_mask = disp.reshape(T, E, CAP)
    exp_counts = cnt.reshape(E)
    return l_aux, combine_weights, dispatch_mask, exp_counts
